# baseline (speedup 1.0000x reference)
import jax
import jax.numpy as jnp
from jax.experimental import pallas as pl
from jax.experimental.pallas import tpu as pltpu

N = 10000
BM = 400
GRID = N // BM


def _pass1(adj_ref, x_ref, W1_ref, b1_ref, W2_ref, g_ref, s_ref):
    @pl.when(pl.program_id(0) == 0)
    def _():
        s_ref[:] = jnp.dot(x_ref[:], W1_ref[:],
                           preferred_element_type=jnp.float32)

    h = jnp.dot(adj_ref[:], s_ref[:], preferred_element_type=jnp.float32)
    h = jnp.maximum(h + b1_ref[:], 0.0)
    g_ref[:] = jnp.dot(h, W2_ref[:], preferred_element_type=jnp.float32)


def _pass2(adj_ref, g_ref, b2_ref, out_ref):
    out_ref[:] = jnp.dot(adj_ref[:], g_ref[:],
                         preferred_element_type=jnp.float32) + b2_ref[:]


def kernel(x, adj, W1, b1, W2, b2):
    nfeat = x.shape[1]
    nhid = W1.shape[1]
    nclass = W2.shape[1]
    b1_2d = b1.reshape(1, nhid)
    b2_2d = b2.reshape(1, nclass)

    adj_spec = pl.BlockSpec((BM, N), lambda i: (i, 0))
    full = lambda shape: pl.BlockSpec(shape, lambda i: (0, 0))

    g = pl.pallas_call(
        _pass1,
        grid=(GRID,),
        in_specs=[
            adj_spec,
            full((N, nfeat)),
            full((nfeat, nhid)),
            full((1, nhid)),
            full((nhid, nclass)),
        ],
        out_specs=pl.BlockSpec((BM, nclass), lambda i: (i, 0)),
        out_shape=jax.ShapeDtypeStruct((N, nclass), jnp.float32),
        scratch_shapes=[pltpu.VMEM((N, nhid), jnp.float32)],
        compiler_params=pltpu.CompilerParams(
            dimension_semantics=("arbitrary",)),
    )(adj, x, W1, b1_2d, W2)

    out = pl.pallas_call(
        _pass2,
        grid=(GRID,),
        in_specs=[adj_spec, full((N, nclass)), full((1, nclass))],
        out_specs=pl.BlockSpec((BM, nclass), lambda i: (i, 0)),
        out_shape=jax.ShapeDtypeStruct((N, nclass), jnp.float32),
        compiler_params=pltpu.CompilerParams(
            dimension_semantics=("arbitrary",)),
    )(adj, g, b2_2d)

    return out


# speedup vs baseline: 1.0241x; 1.0241x over previous
import jax
import jax.numpy as jnp
from jax.experimental import pallas as pl
from jax.experimental.pallas import tpu as pltpu

N = 10000
BM = 400
GRID = N // BM


def _gcn(adj_ref, x_ref, W1_ref, b1_ref, W2_ref, b2_ref, out_ref,
         s_ref, g_ref):
    p = pl.program_id(0)
    i = pl.program_id(1)

    @pl.when((p == 0) & (i == 0))
    def _():
        s_ref[:] = jnp.dot(x_ref[:], W1_ref[:],
                           preferred_element_type=jnp.float32)

    @pl.when(p == 0)
    def _():
        h = jnp.dot(adj_ref[:], s_ref[:],
                    preferred_element_type=jnp.float32)
        h = jnp.maximum(h + b1_ref[:], 0.0)
        g_ref[pl.ds(i * BM, BM), :] = jnp.dot(
            h, W2_ref[:], preferred_element_type=jnp.float32)

    @pl.when(p == 1)
    def _():
        out_ref[:] = jnp.dot(adj_ref[:], g_ref[:],
                             preferred_element_type=jnp.float32) + b2_ref[:]


def kernel(x, adj, W1, b1, W2, b2):
    nfeat = x.shape[1]
    nhid = W1.shape[1]
    nclass = W2.shape[1]
    b1_2d = b1.reshape(1, nhid)
    b2_2d = b2.reshape(1, nclass)

    full = lambda shape: pl.BlockSpec(shape, lambda p, i: (0, 0))

    out = pl.pallas_call(
        _gcn,
        grid=(2, GRID),
        in_specs=[
            pl.BlockSpec((BM, N), lambda p, i: (i, 0)),
            full((N, nfeat)),
            full((nfeat, nhid)),
            full((1, nhid)),
            full((nhid, nclass)),
            full((1, nclass)),
        ],
        out_specs=pl.BlockSpec((BM, nclass), lambda p, i: (p * i, 0)),
        out_shape=jax.ShapeDtypeStruct((N, nclass), jnp.float32),
        scratch_shapes=[
            pltpu.VMEM((N, nhid), jnp.float32),
            pltpu.VMEM((N, nclass), jnp.float32),
        ],
        compiler_params=pltpu.CompilerParams(
            dimension_semantics=("arbitrary", "arbitrary")),
    )(adj, x, W1, b1_2d, W2, b2_2d)

    return out
